# baseline (speedup 1.0000x reference)
import functools

import jax
import jax.numpy as jnp
from jax import lax
from jax.experimental import pallas as pl
from jax.experimental.pallas import tpu as pltpu
from jax.experimental.pallas import tpu_sc as plsc

N = 4096
NC = 2
NS = 16
NW = NC * NS
L = 16
NCHUNK = N // L
RB = 8
NT = 2560
SROWS = (N - NT) // NW
BT = 256


def _prologue_body(sim_ref, p_ref, hp_ref, iou_ref, big_ref, qn_ref,
                   prm_ref):
    p = p_ref[...]
    hp = hp_ref[...]
    iou = iou_ref[...]
    sim = sim_ref[...]
    hn = p + hp - 2.0 * p * hp
    S = 1.0 + jnp.sum(hn * iou)
    logS = jnp.log(S)
    w = jnp.where(hn > 0.0, jnp.log(iou) - logS, 0.0)
    qn = jnp.exp(sim) * hp
    T = jnp.sum(qn)
    cnt = jnp.sum(hp)
    big_ref[0:32, :] = hn
    big_ref[32:64, :] = w
    big_ref[64:65, :] = jnp.full((1, 128), -logS)
    big_ref[65:66, :] = jnp.full((1, 128), T)
    qn_ref[...] = qn
    z = jnp.float32(0.0)
    vals = jnp.stack([-logS, T, cnt, z, z, z, z, z])
    prm_ref[...] = jnp.broadcast_to(vals[:, None], (8, 128))


_prologue = pl.pallas_call(
    _prologue_body,
    out_shape=(
        jax.ShapeDtypeStruct((66, 128), jnp.float32),
        jax.ShapeDtypeStruct((32, 128), jnp.float32),
        jax.ShapeDtypeStruct((8, 128), jnp.float32),
    ),
)


WOFF = N
POFF = 2 * N


def _sc_body(d_hbm, big_hbm, qn_hbm, hp_hbm, out_hbm,
             big_v, qnloc_v, hploc_v, cidx_v, wc_v, ridx_v, rows_v,
             tot_v, sem0, sem1):
    c = lax.axis_index("c")
    s = lax.axis_index("s")
    wid = s * NC + c
    row0 = wid * SROWS

    pltpu.sync_copy(big_hbm, big_v)
    pltpu.sync_copy(qn_hbm.at[pl.ds(row0, SROWS)], qnloc_v.at[pl.ds(0, SROWS)])
    pltpu.sync_copy(hp_hbm.at[pl.ds(row0, SROWS)], hploc_v.at[pl.ds(0, SROWS)])
    mlogs = big_v[pl.ds(POFF, L)]
    t16 = big_v[pl.ds(POFF + 128, L)]

    iota = lax.iota(jnp.int32, L)
    izero = jnp.zeros((L,), jnp.int32)
    fzero = jnp.zeros((L,), jnp.float32)

    scope_cmp = jax.named_scope("sc_compact")
    scope_cmp.__enter__()

    def col_cmp(j, cnt):
        base = j * L
        hn16 = big_v[pl.ds(base, L)]
        m = hn16 > 0.0
        mi = m.astype(jnp.int32)
        offs = jnp.full((L,), cnt, jnp.int32) + plsc.cumsum(mi) - mi
        plsc.store_scatter(cidx_v, [offs], base + iota, mask=m)
        plsc.store_scatter(wc_v, [offs], big_v[pl.ds(WOFF + base, L)], mask=m)
        return cnt + jnp.sum(mi)

    cnt_hn = lax.fori_loop(0, NCHUNK, col_cmp, jnp.int32(0))
    plsc.store_scatter(cidx_v, [cnt_hn + iota], izero, mask=None)
    plsc.store_scatter(wc_v, [cnt_hn + iota], fzero, mask=None)
    nchunk_c = (cnt_hn + L - 1) // L
    npad = nchunk_c * L - cnt_hn
    npad_f = jnp.full((L,), npad, jnp.int32).astype(jnp.float32)

    def row_cmp(j, cnt):
        base = j * L
        hp16 = hploc_v[pl.ds(base, L)]
        m = hp16 > 0.0
        mi = m.astype(jnp.int32)
        offs = jnp.full((L,), cnt, jnp.int32) + plsc.cumsum(mi) - mi
        plsc.store_scatter(ridx_v, [offs], row0 + base + iota, mask=m)
        return cnt + jnp.sum(mi)

    my_cnt = lax.fori_loop(0, SROWS // L, row_cmp, jnp.int32(0))
    plsc.store_scatter(ridx_v, [my_cnt + iota],
                       jnp.full((L,), row0, jnp.int32), mask=None)
    nblk = (my_cnt + RB - 1) // RB
    scope_cmp.__exit__(None, None, None)
    scope_main = jax.named_scope("sc_mainloop")
    scope_main.__enter__()

    def issue(g):
        idxs = ridx_v.at[pl.ds(g * RB, RB)]

        @pl.when(lax.rem(g, 2) == 0)
        def _():
            pltpu.async_copy(d_hbm.at[idxs], rows_v.at[pl.ds(0, RB)], sem0)

        @pl.when(lax.rem(g, 2) == 1)
        def _():
            pltpu.async_copy(d_hbm.at[idxs], rows_v.at[pl.ds(RB, RB)], sem1)

    @pl.when(nblk > 0)
    def _():
        issue(0)

    def blk(g, total):
        par = lax.rem(g, 2)

        @pl.when(g + 1 < nblk)
        def _():
            issue(g + 1)

        @pl.when(par == 0)
        def _():
            pltpu.make_async_copy(
                d_hbm.at[ridx_v.at[pl.ds(0, RB)]],
                rows_v.at[pl.ds(0, RB)], sem0).wait()

        @pl.when(par == 1)
        def _():
            pltpu.make_async_copy(
                d_hbm.at[ridx_v.at[pl.ds(0, RB)]],
                rows_v.at[pl.ds(RB, RB)], sem1).wait()

        rbase = par * RB

        @plsc.parallel_loop(0, nchunk_c * L, step=L, unroll=2,
                            carry=(fzero,) * (2 * RB))
        def accs(base, carry):
            idx16 = cidx_v[pl.ds(base, L)]
            w16 = wc_v[pl.ds(base, L)]
            outs = []
            for r in range(RB):
                e = jnp.exp(
                    plsc.load_gather(
                        rows_v,
                        [jnp.full((L,), rbase + r, jnp.int32), idx16]))
                outs.append(carry[2 * r] + e)
                outs.append(carry[2 * r + 1] + e * w16)
            return tuple(outs)

        e0 = jnp.exp(plsc.load_gather(
            rows_v, [jnp.minimum(iota, RB - 1) + rbase, izero]))
        ra = fzero
        rb = fzero
        for r in range(RB):
            sel = iota == r
            ra = jnp.where(sel, jnp.full((L,), jnp.sum(accs[2 * r])), ra)
            rb = jnp.where(sel, jnp.full((L,), jnp.sum(accs[2 * r + 1])), rb)
        ra = ra - npad_f * e0

        ridx16 = ridx_v[pl.ds(g * RB, L)]
        qn16 = plsc.load_gather(qnloc_v, [ridx16 - row0])
        valid = ((g * RB + iota) < my_cnt) & (iota < RB)
        contrib = (qn16 * mlogs + rb) / (ra + t16)
        return total + jnp.where(valid, contrib, fzero)

    total = lax.fori_loop(0, nblk, blk, fzero)
    scope_main.__exit__(None, None, None)

    tot_v[...] = total
    pltpu.sync_copy(tot_v, out_hbm.at[pl.ds(wid * L, L)])


_sc_main = functools.partial(
    pl.kernel,
    out_type=jax.ShapeDtypeStruct((NW * L,), jnp.float32),
    mesh=plsc.VectorSubcoreMesh(core_axis_name="c", subcore_axis_name="s"),
    compiler_params=pltpu.CompilerParams(needs_layout_passes=False),
    scratch_types=[
        pltpu.VMEM((66 * 128,), jnp.float32),
        pltpu.VMEM((SROWS + L,), jnp.float32),
        pltpu.VMEM((SROWS + L,), jnp.float32),
        pltpu.VMEM((N + L,), jnp.int32),
        pltpu.VMEM((N + L,), jnp.float32),
        pltpu.VMEM((SROWS + L,), jnp.int32),
        pltpu.VMEM((2 * RB, N), jnp.float32),
        pltpu.VMEM((L,), jnp.float32),
        pltpu.SemaphoreType.DMA,
        pltpu.SemaphoreType.DMA,
    ],
)(_sc_body)


def _tc_rows_body(d_ref, w2_ref, qn_ref, hp_ref, prm_ref, out_ref):
    e = jnp.exp(d_ref[...])
    p = jnp.dot(e, w2_ref[...], preferred_element_type=jnp.float32)
    prm = prm_ref[...]
    mlogs = prm[0, 0]
    t = prm[1, 0]
    qn = qn_ref[...]
    hp = hp_ref[...]
    out_ref[...] = hp * (qn * mlogs + p[:, 0]) / (p[:, 1] + t)


_tc_rows = pl.pallas_call(
    _tc_rows_body,
    grid=(NT // BT,),
    in_specs=[
        pl.BlockSpec((BT, N), lambda i: (i, 0)),
        pl.BlockSpec((N, 2), lambda i: (0, 0)),
        pl.BlockSpec((BT,), lambda i: (i,)),
        pl.BlockSpec((BT,), lambda i: (i,)),
        pl.BlockSpec((8, 128), lambda i: (0, 0)),
    ],
    out_specs=pl.BlockSpec((BT,), lambda i: (i,)),
    out_shape=jax.ShapeDtypeStruct((NT,), jnp.float32),
)


def _epilogue_body(psc_ref, ptc_ref, prm_ref, out_ref):
    loss = -(jnp.sum(psc_ref[...]) + jnp.sum(ptc_ref[...]))
    cnt = prm_ref[...][2, 0]
    out_ref[...] = jnp.where(cnt == 0.0, 0.0, loss)[None, None]


_epilogue = pl.pallas_call(
    _epilogue_body,
    out_shape=jax.ShapeDtypeStruct((1, 1), jnp.float32),
)


def kernel(sim_mat, database_sim_mat, pos_mask, hard_pos_mask, neg_mask, iou):
    del neg_mask
    p_f = pos_mask[:, 1].astype(jnp.float32)
    hp_f = hard_pos_mask[:, 1].astype(jnp.float32)
    iou_ = iou[:, 0]

    big, qn, prm = _prologue(
        sim_mat.reshape(32, 128), p_f.reshape(32, 128),
        hp_f.reshape(32, 128), iou_.reshape(32, 128))

    qn1 = qn.reshape(N)

    partials_sc = _sc_main(
        database_sim_mat[NT:], big.reshape(66 * 128), qn1[NT:], hp_f[NT:])

    w2 = jnp.stack([big[32:64].reshape(N), big[0:32].reshape(N)], axis=1)
    partials_tc = _tc_rows(database_sim_mat, w2, qn1[:NT], hp_f[:NT], prm)

    out = _epilogue(partials_sc.reshape(4, 128),
                    partials_tc.reshape(NT // 128, 128), prm)
    return out.reshape(())


# speedup vs baseline: 1.3889x; 1.3889x over previous
import functools

import jax
import jax.numpy as jnp
from jax import lax
from jax.experimental import pallas as pl
from jax.experimental.pallas import tpu as pltpu
from jax.experimental.pallas import tpu_sc as plsc

N = 4096
NC = 2
NS = 16
NW = NC * NS
L = 16
NCHUNK = N // L
RB = 8
NT = 2560
SROWS = (N - NT) // NW
BT = 512


def _prologue_body(sim_ref, p_ref, hp_ref, iou_ref, big_ref, qn_ref,
                   prm_ref):
    p = p_ref[...]
    hp = hp_ref[...]
    iou = iou_ref[...]
    sim = sim_ref[...]
    hn = p + hp - 2.0 * p * hp
    S = 1.0 + jnp.sum(hn * iou)
    logS = jnp.log(S)
    w = jnp.where(hn > 0.0, jnp.log(iou) - logS, 0.0)
    qn = jnp.exp(sim) * hp
    T = jnp.sum(qn)
    cnt = jnp.sum(hp)
    big_ref[0:32, :] = hn
    big_ref[32:64, :] = w
    big_ref[64:65, :] = jnp.full((1, 128), -logS)
    big_ref[65:66, :] = jnp.full((1, 128), T)
    qn_ref[...] = qn
    z = jnp.float32(0.0)
    vals = jnp.stack([-logS, T, cnt, z, z, z, z, z])
    prm_ref[...] = jnp.broadcast_to(vals[:, None], (8, 128))


_prologue = pl.pallas_call(
    _prologue_body,
    out_shape=(
        jax.ShapeDtypeStruct((66, 128), jnp.float32),
        jax.ShapeDtypeStruct((32, 128), jnp.float32),
        jax.ShapeDtypeStruct((8, 128), jnp.float32),
    ),
)


WOFF = N
POFF = 2 * N


def _sc_body(d_hbm, big_hbm, qn_hbm, hp_hbm, out_hbm,
             big_v, qnloc_v, hploc_v, cidx_v, wc_v, ridx_v, rows_v,
             tot_v, sem0, sem1):
    c = lax.axis_index("c")
    s = lax.axis_index("s")
    wid = s * NC + c
    row0 = NT + wid * SROWS

    pltpu.sync_copy(big_hbm, big_v)
    pltpu.sync_copy(qn_hbm.at[pl.ds(row0, SROWS)], qnloc_v.at[pl.ds(0, SROWS)])
    pltpu.sync_copy(hp_hbm.at[pl.ds(row0, SROWS)], hploc_v.at[pl.ds(0, SROWS)])
    mlogs = big_v[pl.ds(POFF, L)]
    t16 = big_v[pl.ds(POFF + 128, L)]

    iota = lax.iota(jnp.int32, L)
    izero = jnp.zeros((L,), jnp.int32)
    fzero = jnp.zeros((L,), jnp.float32)

    scope_cmp = jax.named_scope("sc_compact")
    scope_cmp.__enter__()

    def col_cmp(j, cnt):
        base = j * L
        hn16 = big_v[pl.ds(base, L)]
        m = hn16 > 0.0
        mi = m.astype(jnp.int32)
        offs = jnp.full((L,), cnt, jnp.int32) + plsc.cumsum(mi) - mi
        plsc.store_scatter(cidx_v, [offs], base + iota, mask=m)
        plsc.store_scatter(wc_v, [offs], big_v[pl.ds(WOFF + base, L)], mask=m)
        return cnt + jnp.sum(mi)

    cnt_hn = lax.fori_loop(0, NCHUNK, col_cmp, jnp.int32(0))
    plsc.store_scatter(cidx_v, [cnt_hn + iota], izero, mask=None)
    plsc.store_scatter(wc_v, [cnt_hn + iota], fzero, mask=None)
    nchunk_c = (cnt_hn + L - 1) // L
    npad = nchunk_c * L - cnt_hn
    npad_f = jnp.full((L,), npad, jnp.int32).astype(jnp.float32)

    def row_cmp(j, cnt):
        base = j * L
        hp16 = hploc_v[pl.ds(base, L)]
        m = hp16 > 0.0
        mi = m.astype(jnp.int32)
        offs = jnp.full((L,), cnt, jnp.int32) + plsc.cumsum(mi) - mi
        plsc.store_scatter(ridx_v, [offs], row0 + base + iota, mask=m)
        return cnt + jnp.sum(mi)

    my_cnt = lax.fori_loop(0, SROWS // L, row_cmp, jnp.int32(0))
    plsc.store_scatter(ridx_v, [my_cnt + iota],
                       jnp.full((L,), row0, jnp.int32), mask=None)
    nblk = (my_cnt + RB - 1) // RB
    scope_cmp.__exit__(None, None, None)
    scope_main = jax.named_scope("sc_mainloop")
    scope_main.__enter__()

    def issue(g):
        idxs = ridx_v.at[pl.ds(g * RB, RB)]

        @pl.when(lax.rem(g, 2) == 0)
        def _():
            pltpu.async_copy(d_hbm.at[idxs], rows_v.at[pl.ds(0, RB)], sem0)

        @pl.when(lax.rem(g, 2) == 1)
        def _():
            pltpu.async_copy(d_hbm.at[idxs], rows_v.at[pl.ds(RB, RB)], sem1)

    @pl.when(nblk > 0)
    def _():
        issue(0)

    def blk(g, total):
        par = lax.rem(g, 2)

        @pl.when(g + 1 < nblk)
        def _():
            issue(g + 1)

        @pl.when(par == 0)
        def _():
            pltpu.make_async_copy(
                d_hbm.at[ridx_v.at[pl.ds(0, RB)]],
                rows_v.at[pl.ds(0, RB)], sem0).wait()

        @pl.when(par == 1)
        def _():
            pltpu.make_async_copy(
                d_hbm.at[ridx_v.at[pl.ds(0, RB)]],
                rows_v.at[pl.ds(RB, RB)], sem1).wait()

        rbase = par * RB

        @plsc.parallel_loop(0, nchunk_c * L, step=L, unroll=2,
                            carry=(fzero,) * (2 * RB))
        def accs(base, carry):
            idx16 = cidx_v[pl.ds(base, L)]
            w16 = wc_v[pl.ds(base, L)]
            outs = []
            for r in range(RB):
                e = jnp.exp(
                    plsc.load_gather(
                        rows_v,
                        [jnp.full((L,), rbase + r, jnp.int32), idx16]))
                outs.append(carry[2 * r] + e)
                outs.append(carry[2 * r + 1] + e * w16)
            return tuple(outs)

        e0 = jnp.exp(plsc.load_gather(
            rows_v, [jnp.minimum(iota, RB - 1) + rbase, izero]))
        ra = fzero
        rb = fzero
        for r in range(RB):
            sel = iota == r
            ra = jnp.where(sel, jnp.full((L,), jnp.sum(accs[2 * r])), ra)
            rb = jnp.where(sel, jnp.full((L,), jnp.sum(accs[2 * r + 1])), rb)
        ra = ra - npad_f * e0

        ridx16 = ridx_v[pl.ds(g * RB, L)]
        qn16 = plsc.load_gather(qnloc_v, [ridx16 - row0])
        valid = ((g * RB + iota) < my_cnt) & (iota < RB)
        contrib = (qn16 * mlogs + rb) / (ra + t16)
        return total + jnp.where(valid, contrib, fzero)

    total = lax.fori_loop(0, nblk, blk, fzero)
    scope_main.__exit__(None, None, None)

    tot_v[...] = total
    pltpu.sync_copy(tot_v, out_hbm.at[pl.ds(wid * L, L)])


_sc_main = functools.partial(
    pl.kernel,
    out_type=jax.ShapeDtypeStruct((NW * L,), jnp.float32),
    mesh=plsc.VectorSubcoreMesh(core_axis_name="c", subcore_axis_name="s"),
    compiler_params=pltpu.CompilerParams(needs_layout_passes=False),
    scratch_types=[
        pltpu.VMEM((66 * 128,), jnp.float32),
        pltpu.VMEM((SROWS + L,), jnp.float32),
        pltpu.VMEM((SROWS + L,), jnp.float32),
        pltpu.VMEM((N + L,), jnp.int32),
        pltpu.VMEM((N + L,), jnp.float32),
        pltpu.VMEM((SROWS + L,), jnp.int32),
        pltpu.VMEM((2 * RB, N), jnp.float32),
        pltpu.VMEM((L,), jnp.float32),
        pltpu.SemaphoreType.DMA,
        pltpu.SemaphoreType.DMA,
    ],
)(_sc_body)


def _tc_rows_body(d_ref, w2_ref, qn_ref, hp_ref, prm_ref, out_ref):
    e = jnp.exp(d_ref[...])
    p = jnp.dot(e, w2_ref[...], preferred_element_type=jnp.float32)
    prm = prm_ref[...]
    mlogs = prm[0, 0]
    t = prm[1, 0]
    qn = qn_ref[...]
    hp = hp_ref[...]
    out_ref[...] = hp * (qn * mlogs + p[:, 0]) / (p[:, 1] + t)


_tc_rows = pl.pallas_call(
    _tc_rows_body,
    grid=(NT // BT,),
    in_specs=[
        pl.BlockSpec((BT, N), lambda i: (i, 0)),
        pl.BlockSpec((N, 2), lambda i: (0, 0)),
        pl.BlockSpec((BT,), lambda i: (i,)),
        pl.BlockSpec((BT,), lambda i: (i,)),
        pl.BlockSpec((8, 128), lambda i: (0, 0)),
    ],
    out_specs=pl.BlockSpec((BT,), lambda i: (i,)),
    out_shape=jax.ShapeDtypeStruct((NT,), jnp.float32),
)


def _epilogue_body(psc_ref, ptc_ref, prm_ref, out_ref):
    loss = -(jnp.sum(psc_ref[...]) + jnp.sum(ptc_ref[...]))
    cnt = prm_ref[...][2, 0]
    out_ref[...] = jnp.where(cnt == 0.0, 0.0, loss)[None, None]


_epilogue = pl.pallas_call(
    _epilogue_body,
    out_shape=jax.ShapeDtypeStruct((1, 1), jnp.float32),
)


def kernel(sim_mat, database_sim_mat, pos_mask, hard_pos_mask, neg_mask, iou):
    del neg_mask
    p_f = pos_mask[:, 1].astype(jnp.float32)
    hp_f = hard_pos_mask[:, 1].astype(jnp.float32)
    iou_ = iou[:, 0]

    big, qn, prm = _prologue(
        sim_mat.reshape(32, 128), p_f.reshape(32, 128),
        hp_f.reshape(32, 128), iou_.reshape(32, 128))

    qn1 = qn.reshape(N)

    partials_sc = _sc_main(
        database_sim_mat, big.reshape(66 * 128), qn1, hp_f)

    w2 = jnp.stack([big[32:64].reshape(N), big[0:32].reshape(N)], axis=1)
    partials_tc = _tc_rows(database_sim_mat, w2, qn1[:NT], hp_f[:NT], prm)

    out = _epilogue(partials_sc.reshape(4, 128),
                    partials_tc.reshape(NT // 128, 128), prm)
    return out.reshape(())


# speedup vs baseline: 1.4296x; 1.0293x over previous
import functools

import jax
import jax.numpy as jnp
from jax import lax
from jax.experimental import pallas as pl
from jax.experimental.pallas import tpu as pltpu
from jax.experimental.pallas import tpu_sc as plsc

N = 4096
NC = 2
NS = 16
NW = NC * NS
L = 16
NCHUNK = N // L
RB = 8
NT = 3072
SROWS = (N - NT) // NW
BT = 512


def _prologue_body(sim_ref, p_ref, hp_ref, iou_ref, big_ref, qn_ref,
                   prm_ref):
    p = p_ref[...]
    hp = hp_ref[...]
    iou = iou_ref[...]
    sim = sim_ref[...]
    hn = p + hp - 2.0 * p * hp
    S = 1.0 + jnp.sum(hn * iou)
    logS = jnp.log(S)
    w = jnp.where(hn > 0.0, jnp.log(iou) - logS, 0.0)
    qn = jnp.exp(sim) * hp
    T = jnp.sum(qn)
    cnt = jnp.sum(hp)
    big_ref[0:32, :] = hn
    big_ref[32:64, :] = w
    big_ref[64:65, :] = jnp.full((1, 128), -logS)
    big_ref[65:66, :] = jnp.full((1, 128), T)
    qn_ref[...] = qn
    z = jnp.float32(0.0)
    vals = jnp.stack([-logS, T, cnt, z, z, z, z, z])
    prm_ref[...] = jnp.broadcast_to(vals[:, None], (8, 128))


_prologue = pl.pallas_call(
    _prologue_body,
    out_shape=(
        jax.ShapeDtypeStruct((66, 128), jnp.float32),
        jax.ShapeDtypeStruct((32, 128), jnp.float32),
        jax.ShapeDtypeStruct((8, 128), jnp.float32),
    ),
)


WOFF = N
POFF = 2 * N


def _sc_body(d_hbm, big_hbm, qn_hbm, hp_hbm, out_hbm,
             big_v, qnloc_v, hploc_v, cidx_v, wc_v, ridx_v, rows_v,
             tot_v, sem0, sem1):
    c = lax.axis_index("c")
    s = lax.axis_index("s")
    wid = s * NC + c
    row0 = NT + wid * SROWS

    pltpu.sync_copy(big_hbm, big_v)
    pltpu.sync_copy(qn_hbm.at[pl.ds(row0, SROWS)], qnloc_v.at[pl.ds(0, SROWS)])
    pltpu.sync_copy(hp_hbm.at[pl.ds(row0, SROWS)], hploc_v.at[pl.ds(0, SROWS)])
    mlogs = big_v[pl.ds(POFF, L)]
    t16 = big_v[pl.ds(POFF + 128, L)]

    iota = lax.iota(jnp.int32, L)
    izero = jnp.zeros((L,), jnp.int32)
    fzero = jnp.zeros((L,), jnp.float32)

    scope_cmp = jax.named_scope("sc_compact")
    scope_cmp.__enter__()

    def col_cmp(j, cnt):
        base = j * L
        hn16 = big_v[pl.ds(base, L)]
        m = hn16 > 0.0
        mi = m.astype(jnp.int32)
        offs = jnp.full((L,), cnt, jnp.int32) + plsc.cumsum(mi) - mi
        plsc.store_scatter(cidx_v, [offs], base + iota, mask=m)
        plsc.store_scatter(wc_v, [offs], big_v[pl.ds(WOFF + base, L)], mask=m)
        return cnt + jnp.sum(mi)

    cnt_hn = lax.fori_loop(0, NCHUNK, col_cmp, jnp.int32(0))
    plsc.store_scatter(cidx_v, [cnt_hn + iota], izero, mask=None)
    plsc.store_scatter(wc_v, [cnt_hn + iota], fzero, mask=None)
    nchunk_c = (cnt_hn + L - 1) // L
    npad = nchunk_c * L - cnt_hn
    npad_f = jnp.full((L,), npad, jnp.int32).astype(jnp.float32)

    def row_cmp(j, cnt):
        base = j * L
        hp16 = hploc_v[pl.ds(base, L)]
        m = hp16 > 0.0
        mi = m.astype(jnp.int32)
        offs = jnp.full((L,), cnt, jnp.int32) + plsc.cumsum(mi) - mi
        plsc.store_scatter(ridx_v, [offs], row0 + base + iota, mask=m)
        return cnt + jnp.sum(mi)

    my_cnt = lax.fori_loop(0, SROWS // L, row_cmp, jnp.int32(0))
    plsc.store_scatter(ridx_v, [my_cnt + iota],
                       jnp.full((L,), row0, jnp.int32), mask=None)
    nblk = (my_cnt + RB - 1) // RB
    scope_cmp.__exit__(None, None, None)
    scope_main = jax.named_scope("sc_mainloop")
    scope_main.__enter__()

    def issue(g):
        idxs = ridx_v.at[pl.ds(g * RB, RB)]

        @pl.when(lax.rem(g, 2) == 0)
        def _():
            pltpu.async_copy(d_hbm.at[idxs], rows_v.at[pl.ds(0, RB)], sem0)

        @pl.when(lax.rem(g, 2) == 1)
        def _():
            pltpu.async_copy(d_hbm.at[idxs], rows_v.at[pl.ds(RB, RB)], sem1)

    @pl.when(nblk > 0)
    def _():
        issue(0)

    def blk(g, total):
        par = lax.rem(g, 2)

        @pl.when(g + 1 < nblk)
        def _():
            issue(g + 1)

        @pl.when(par == 0)
        def _():
            pltpu.make_async_copy(
                d_hbm.at[ridx_v.at[pl.ds(0, RB)]],
                rows_v.at[pl.ds(0, RB)], sem0).wait()

        @pl.when(par == 1)
        def _():
            pltpu.make_async_copy(
                d_hbm.at[ridx_v.at[pl.ds(0, RB)]],
                rows_v.at[pl.ds(RB, RB)], sem1).wait()

        rbase = par * RB

        @plsc.parallel_loop(0, nchunk_c * L, step=L, unroll=2,
                            carry=(fzero,) * (2 * RB))
        def accs(base, carry):
            idx16 = cidx_v[pl.ds(base, L)]
            w16 = wc_v[pl.ds(base, L)]
            outs = []
            for r in range(RB):
                e = jnp.exp(
                    plsc.load_gather(
                        rows_v,
                        [jnp.full((L,), rbase + r, jnp.int32), idx16]))
                outs.append(carry[2 * r] + e)
                outs.append(carry[2 * r + 1] + e * w16)
            return tuple(outs)

        e0 = jnp.exp(plsc.load_gather(
            rows_v, [jnp.minimum(iota, RB - 1) + rbase, izero]))
        ra = fzero
        rb = fzero
        for r in range(RB):
            sel = iota == r
            ra = jnp.where(sel, jnp.full((L,), jnp.sum(accs[2 * r])), ra)
            rb = jnp.where(sel, jnp.full((L,), jnp.sum(accs[2 * r + 1])), rb)
        ra = ra - npad_f * e0

        ridx16 = ridx_v[pl.ds(g * RB, L)]
        qn16 = plsc.load_gather(qnloc_v, [ridx16 - row0])
        valid = ((g * RB + iota) < my_cnt) & (iota < RB)
        contrib = (qn16 * mlogs + rb) / (ra + t16)
        return total + jnp.where(valid, contrib, fzero)

    total = lax.fori_loop(0, nblk, blk, fzero)
    scope_main.__exit__(None, None, None)

    tot_v[...] = total
    pltpu.sync_copy(tot_v, out_hbm.at[pl.ds(wid * L, L)])


_sc_main = functools.partial(
    pl.kernel,
    out_type=jax.ShapeDtypeStruct((NW * L,), jnp.float32),
    mesh=plsc.VectorSubcoreMesh(core_axis_name="c", subcore_axis_name="s"),
    compiler_params=pltpu.CompilerParams(needs_layout_passes=False),
    scratch_types=[
        pltpu.VMEM((66 * 128,), jnp.float32),
        pltpu.VMEM((SROWS + L,), jnp.float32),
        pltpu.VMEM((SROWS + L,), jnp.float32),
        pltpu.VMEM((N + L,), jnp.int32),
        pltpu.VMEM((N + L,), jnp.float32),
        pltpu.VMEM((SROWS + L,), jnp.int32),
        pltpu.VMEM((2 * RB, N), jnp.float32),
        pltpu.VMEM((L,), jnp.float32),
        pltpu.SemaphoreType.DMA,
        pltpu.SemaphoreType.DMA,
    ],
)(_sc_body)


def _tc_rows_body(d_ref, w2_ref, qn_ref, hp_ref, prm_ref, out_ref):
    e = jnp.exp(d_ref[...])
    p = jnp.dot(e, w2_ref[...], preferred_element_type=jnp.float32)
    prm = prm_ref[...]
    mlogs = prm[0, 0]
    t = prm[1, 0]
    qn = qn_ref[...]
    hp = hp_ref[...]
    out_ref[...] = hp * (qn * mlogs + p[:, 0]) / (p[:, 1] + t)


_tc_rows = pl.pallas_call(
    _tc_rows_body,
    grid=(NT // BT,),
    in_specs=[
        pl.BlockSpec((BT, N), lambda i: (i, 0)),
        pl.BlockSpec((N, 2), lambda i: (0, 0)),
        pl.BlockSpec((BT,), lambda i: (i,)),
        pl.BlockSpec((BT,), lambda i: (i,)),
        pl.BlockSpec((8, 128), lambda i: (0, 0)),
    ],
    out_specs=pl.BlockSpec((BT,), lambda i: (i,)),
    out_shape=jax.ShapeDtypeStruct((NT,), jnp.float32),
)


def _epilogue_body(psc_ref, ptc_ref, prm_ref, out_ref):
    loss = -(jnp.sum(psc_ref[...]) + jnp.sum(ptc_ref[...]))
    cnt = prm_ref[...][2, 0]
    out_ref[...] = jnp.where(cnt == 0.0, 0.0, loss)[None, None]


_epilogue = pl.pallas_call(
    _epilogue_body,
    out_shape=jax.ShapeDtypeStruct((1, 1), jnp.float32),
)


def kernel(sim_mat, database_sim_mat, pos_mask, hard_pos_mask, neg_mask, iou):
    del neg_mask
    p_f = pos_mask[:, 1].astype(jnp.float32)
    hp_f = hard_pos_mask[:, 1].astype(jnp.float32)
    iou_ = iou[:, 0]

    big, qn, prm = _prologue(
        sim_mat.reshape(32, 128), p_f.reshape(32, 128),
        hp_f.reshape(32, 128), iou_.reshape(32, 128))

    qn1 = qn.reshape(N)

    partials_sc = _sc_main(
        database_sim_mat, big.reshape(66 * 128), qn1, hp_f)

    w2 = jnp.stack([big[32:64].reshape(N), big[0:32].reshape(N)], axis=1)
    partials_tc = _tc_rows(database_sim_mat, w2, qn1[:NT], hp_f[:NT], prm)

    out = _epilogue(partials_sc.reshape(4, 128),
                    partials_tc.reshape(NT // 128, 128), prm)
    return out.reshape(())
